# baseline (speedup 1.0000x reference)
import functools

import jax
import jax.numpy as jnp
import numpy as np
from jax import lax
from jax.experimental import pallas as pl
from jax.experimental.pallas import tpu as pltpu
from jax.experimental.pallas import tpu_sc as plsc

_N = 10000
_E = 320000
_F = 128
_G = 50
_CUTOFF = 6.0

_NUM_CORES = 2
_NUM_SUBCORES = 16
_NUM_TILES = _NUM_CORES * _NUM_SUBCORES
_EDGES_PER_TILE = _E // _NUM_TILES
_CHUNK = 40
_NCHUNKS = _EDGES_PER_TILE // _CHUNK
_ROWS_PER_TILE = 624
_ROWS_REMAINDER = _N - _NUM_SUBCORES * _ROWS_PER_TILE


def _ssp(v):
    return jnp.maximum(v, 0.0) + jnp.log1p(jnp.exp(-jnp.abs(v))) - jnp.log(2.0)


def _h_body(x_ref, w_ref, o_ref):
    o_ref[...] = jnp.dot(x_ref[...], w_ref[...],
                         preferred_element_type=jnp.float32)


def _compute_h(x, lin1_w):
    nb = 10
    bn = _N // nb
    return pl.pallas_call(
        _h_body,
        grid=(nb,),
        in_specs=[
            pl.BlockSpec((bn, _F), lambda i: (i, 0)),
            pl.BlockSpec((_F, _F), lambda i: (0, 0)),
        ],
        out_specs=pl.BlockSpec((bn, _F), lambda i: (i, 0)),
        out_shape=jax.ShapeDtypeStruct((_N, _F), jnp.float32),
    )(x, lin1_w)


def _cutoff_body(ew_ref, o_ref):
    o_ref[...] = 0.5 * (jnp.cos(ew_ref[...] * (jnp.pi / _CUTOFF)) + 1.0)


def _compute_cutoff(edge_weight):
    ew2 = edge_weight.reshape(_E // _F, _F)
    c = pl.pallas_call(
        _cutoff_body,
        out_shape=jax.ShapeDtypeStruct((_E // _F, _F), jnp.float32),
    )(ew2)
    return c.reshape(_E, 1)


def _filter_body(ea_ref, c_ref, w1_ref, b1_ref, w2_ref, b2_ref, o_ref):
    a = jnp.dot(ea_ref[...], w1_ref[...], preferred_element_type=jnp.float32)
    a = a + b1_ref[...]
    f = jnp.dot(_ssp(a), w2_ref[...], preferred_element_type=jnp.float32)
    f = f + b2_ref[...]
    f = f * c_ref[...]
    lo = jax.lax.bitcast_convert_type(f[:, : _F // 2], jnp.int32)
    hi = jax.lax.bitcast_convert_type(f[:, _F // 2:], jnp.int32)
    lo16 = jax.lax.shift_right_logical(lo + 0x8000, 16)
    hi16 = (hi + 0x8000) & jnp.int32(-65536)
    o_ref[...] = lo16 | hi16


def _compute_wf(edge_attr, cutoff, mlp_w1, mlp_b1, mlp_w2, mlp_b2):
    be = 2000
    nb = _E // be
    return pl.pallas_call(
        _filter_body,
        grid=(nb,),
        in_specs=[
            pl.BlockSpec((be, _G), lambda i: (i, 0)),
            pl.BlockSpec((be, 1), lambda i: (i, 0)),
            pl.BlockSpec((_G, _F), lambda i: (0, 0)),
            pl.BlockSpec((1, _F), lambda i: (0, 0)),
            pl.BlockSpec((_F, _F), lambda i: (0, 0)),
            pl.BlockSpec((1, _F), lambda i: (0, 0)),
        ],
        out_specs=pl.BlockSpec((be, _F // 2), lambda i: (i, 0)),
        out_shape=jax.ShapeDtypeStruct((_E, _F // 2), jnp.int32),
    )(edge_attr, cutoff, mlp_w1, mlp_b1.reshape(1, _F), mlp_w2,
      mlp_b2.reshape(1, _F))


@functools.partial(
    pl.kernel,
    out_type=jax.ShapeDtypeStruct((_NUM_CORES, _N, _F), jnp.float32),
    mesh=plsc.VectorSubcoreMesh(core_axis_name="c", subcore_axis_name="s"),
    scratch_types=[
        pltpu.VMEM((_EDGES_PER_TILE,), jnp.int32),
        [pltpu.VMEM((_CHUNK,), jnp.int32)] * 2,
        [pltpu.VMEM((_CHUNK, _F), jnp.float32)] * 2,
        [pltpu.VMEM((_CHUNK, _F // 2), jnp.int32)] * 2,
        pltpu.VMEM_SHARED((_N, _F), jnp.float32),
        [pltpu.SemaphoreType.DMA] * 2,
        [pltpu.SemaphoreType.DMA] * 2,
        [pltpu.SemaphoreType.DMA] * 2,
    ],
)
def _sc_message_pass(h_hbm, src_hbm, dst_hbm, wf_hbm, zeros_hbm, out_hbm,
                     src_all, dst_v, rows_v, wf_v, acc,
                     semi, semg, sems):
    c = lax.axis_index("c")
    s = lax.axis_index("s")
    wid = c * _NUM_SUBCORES + s
    tile_base = wid * _EDGES_PER_TILE

    stripe = pl.ds(s * _ROWS_PER_TILE, _ROWS_PER_TILE)
    rem = pl.ds(_NUM_SUBCORES * _ROWS_PER_TILE, _ROWS_REMAINDER)
    pltpu.sync_copy(zeros_hbm.at[stripe], acc.at[stripe])

    @pl.when(s == 0)
    def _():
        pltpu.sync_copy(zeros_hbm.at[rem], acc.at[rem])

    pltpu.sync_copy(src_hbm.at[pl.ds(tile_base, _EDGES_PER_TILE)], src_all)
    plsc.subcore_barrier()

    def idx_copy(j, slot):
        return pltpu.make_async_copy(
            dst_hbm.at[pl.ds(tile_base + j * _CHUNK, _CHUNK)],
            dst_v[slot], semi[slot])

    def gather_copy(j, slot):
        return pltpu.make_async_copy(
            h_hbm.at[src_all.at[pl.ds(j * _CHUNK, _CHUNK)]],
            rows_v[slot], semg[slot])

    def wf_copy(j, slot):
        return pltpu.make_async_copy(
            wf_hbm.at[pl.ds(tile_base + j * _CHUNK, _CHUNK)],
            wf_v[slot], semg[slot])

    def scatter_copy(slot):
        return pltpu.make_async_copy(
            rows_v[slot], acc.at[dst_v[slot]], sems[slot])

    def issue(j, slot):
        idx_copy(j, slot).start()
        gather_copy(j, slot).start()
        wf_copy(j, slot).start()

    def multiply(slot):
        mask = jnp.full((16,), -65536, jnp.int32)
        sixteen = jnp.full((16,), 16, jnp.int32)

        def mul_row(r, carry2):
            for g in range(_F // 32):
                ww = wf_v[slot][r, pl.ds(g * 16, 16)]
                wlo = jax.lax.bitcast_convert_type(ww << sixteen,
                                                   jnp.float32)
                whi = jax.lax.bitcast_convert_type(ww & mask, jnp.float32)
                sl_lo = pl.ds(g * 16, 16)
                sl_hi = pl.ds(g * 16 + _F // 2, 16)
                rows_v[slot][r, sl_lo] = rows_v[slot][r, sl_lo] * wlo
                rows_v[slot][r, sl_hi] = rows_v[slot][r, sl_hi] * whi
            return carry2

        lax.fori_loop(0, _CHUNK, mul_row, 0, unroll=False)

    def finish(j, slot):
        gather_copy(j, slot).wait()
        wf_copy(j, slot).wait()
        multiply(slot)
        idx_copy(j, slot).wait()
        pltpu.async_copy(rows_v[slot], acc.at[dst_v[slot]], sems[slot],
                         add=True)

    issue(0, 0)

    def pair_body(i, carry):
        for b in range(2):
            j = 2 * i + b
            @pl.when(j > 0)
            def _():
                scatter_copy(1 - b).wait()
            issue_j = j + 1
            idx_copy(issue_j, 1 - b).start()
            gather_copy(issue_j, 1 - b).start()
            wf_copy(issue_j, 1 - b).start()
            finish(j, b)
        return carry

    lax.fori_loop(0, _NCHUNKS // 2 - 1, pair_body, 0, unroll=False)
    scatter_copy(1).wait()
    idx_copy(_NCHUNKS - 1, 1).start()
    gather_copy(_NCHUNKS - 1, 1).start()
    wf_copy(_NCHUNKS - 1, 1).start()
    finish(_NCHUNKS - 2, 0)
    finish(_NCHUNKS - 1, 1)
    scatter_copy(0).wait()
    scatter_copy(1).wait()

    plsc.subcore_barrier()
    pltpu.sync_copy(acc.at[stripe], out_hbm.at[c, stripe])

    @pl.when(s == 0)
    def _():
        pltpu.sync_copy(acc.at[rem], out_hbm.at[c, rem])


def _out_body(p0_ref, p1_ref, w2_ref, b2_ref, w_ref, b_ref, o_ref):
    agg = p0_ref[...] + p1_ref[...]
    h2 = jnp.dot(agg, w2_ref[...], preferred_element_type=jnp.float32)
    h2 = h2 + b2_ref[...]
    o_ref[...] = jnp.dot(_ssp(h2), w_ref[...],
                         preferred_element_type=jnp.float32) + b_ref[...]


def _compute_out(p0, p1, lin2_w, lin2_b, lin_w, lin_b):
    nb = 10
    bn = _N // nb
    return pl.pallas_call(
        _out_body,
        grid=(nb,),
        in_specs=[
            pl.BlockSpec((bn, _F), lambda i: (i, 0)),
            pl.BlockSpec((bn, _F), lambda i: (i, 0)),
            pl.BlockSpec((_F, _F), lambda i: (0, 0)),
            pl.BlockSpec((1, _F), lambda i: (0, 0)),
            pl.BlockSpec((_F, _F), lambda i: (0, 0)),
            pl.BlockSpec((1, _F), lambda i: (0, 0)),
        ],
        out_specs=pl.BlockSpec((bn, _F), lambda i: (i, 0)),
        out_shape=jax.ShapeDtypeStruct((_N, _F), jnp.float32),
    )(p0, p1, lin2_w, lin2_b.reshape(1, _F), lin_w, lin_b.reshape(1, _F))


def kernel(x, edge_index, edge_weight, edge_attr, output,
           mlp_w1, mlp_b1, mlp_w2, mlp_b2,
           lin1_w, lin2_w, lin2_b, lin_w, lin_b):
    h = _compute_h(x, lin1_w)
    cutoff = _compute_cutoff(edge_weight)
    wf = _compute_wf(edge_attr, cutoff, mlp_w1, mlp_b1, mlp_w2, mlp_b2)
    src = edge_index[0]
    dst = edge_index[1]
    partials = _sc_message_pass(h, src, dst, wf, output)
    return _compute_out(partials[0], partials[1], lin2_w, lin2_b,
                        lin_w, lin_b)


# speedup vs baseline: 1.1324x; 1.1324x over previous
import functools

import jax
import jax.numpy as jnp
import numpy as np
from jax import lax
from jax.experimental import pallas as pl
from jax.experimental.pallas import tpu as pltpu
from jax.experimental.pallas import tpu_sc as plsc

_N = 10000
_E = 320000
_F = 128
_G = 50
_CUTOFF = 6.0

_NUM_CORES = 2
_NUM_SUBCORES = 16
_NUM_TILES = _NUM_CORES * _NUM_SUBCORES
_EDGES_PER_TILE = _E // _NUM_TILES
_CHUNK = 40
_NCHUNKS = _EDGES_PER_TILE // _CHUNK
_ROWS_PER_TILE = 624
_ROWS_REMAINDER = _N - _NUM_SUBCORES * _ROWS_PER_TILE


def _ssp(v):
    return jnp.maximum(v, 0.0) + jnp.log1p(jnp.exp(-jnp.abs(v))) - jnp.log(2.0)


def _h_body(x_ref, w_ref, o_ref):
    o_ref[...] = jnp.dot(x_ref[...], w_ref[...],
                         preferred_element_type=jnp.float32)


def _compute_h(x, lin1_w):
    nb = 10
    bn = _N // nb
    return pl.pallas_call(
        _h_body,
        grid=(nb,),
        in_specs=[
            pl.BlockSpec((bn, _F), lambda i: (i, 0)),
            pl.BlockSpec((_F, _F), lambda i: (0, 0)),
        ],
        out_specs=pl.BlockSpec((bn, _F), lambda i: (i, 0)),
        out_shape=jax.ShapeDtypeStruct((_N, _F), jnp.float32),
    )(x, lin1_w)


def _cutoff_body(ew_ref, o_ref):
    o_ref[...] = 0.5 * (jnp.cos(ew_ref[...] * (jnp.pi / _CUTOFF)) + 1.0)


def _compute_cutoff(edge_weight):
    ew2 = edge_weight.reshape(_E // _F, _F)
    c = pl.pallas_call(
        _cutoff_body,
        out_shape=jax.ShapeDtypeStruct((_E // _F, _F), jnp.float32),
    )(ew2)
    return jnp.broadcast_to(c.reshape(_E, 1), (_E, 8))


def _filter_body(ea_ref, c_ref, w1_ref, b1_ref, w2_ref, b2_ref, o_ref):
    a = jax.lax.dot_general(ea_ref[...], w1_ref[...],
                            (((0,), (0,)), ((), ())),
                            preferred_element_type=jnp.float32)
    a = a + b1_ref[...]
    f = jnp.dot(_ssp(a), w2_ref[...], preferred_element_type=jnp.float32)
    f = f + b2_ref[...]
    o_ref[...] = f * c_ref[...][:, 0:1]


def _compute_wf(edge_attr_t, cutoff8, mlp_w1, mlp_b1, mlp_w2, mlp_b2):
    be = 2560
    nb = _E // be
    return pl.pallas_call(
        _filter_body,
        grid=(nb,),
        in_specs=[
            pl.BlockSpec((_G, be), lambda i: (0, i)),
            pl.BlockSpec((be, 8), lambda i: (i, 0)),
            pl.BlockSpec((_G, _F), lambda i: (0, 0)),
            pl.BlockSpec((1, _F), lambda i: (0, 0)),
            pl.BlockSpec((_F, _F), lambda i: (0, 0)),
            pl.BlockSpec((1, _F), lambda i: (0, 0)),
        ],
        out_specs=pl.BlockSpec((be, _F), lambda i: (i, 0)),
        out_shape=jax.ShapeDtypeStruct((_E, _F), jnp.float32),
    )(edge_attr_t, cutoff8, mlp_w1, mlp_b1.reshape(1, _F), mlp_w2,
      mlp_b2.reshape(1, _F))


@functools.partial(
    pl.kernel,
    out_type=jax.ShapeDtypeStruct((_NUM_CORES, _N, _F), jnp.float32),
    mesh=plsc.VectorSubcoreMesh(core_axis_name="c", subcore_axis_name="s"),
    scratch_types=[
        pltpu.VMEM((_EDGES_PER_TILE,), jnp.int32),
        [pltpu.VMEM((_CHUNK,), jnp.int32)] * 2,
        [pltpu.VMEM((_CHUNK, _F), jnp.float32)] * 2,
        [pltpu.VMEM((_CHUNK, _F), jnp.float32)] * 2,
        pltpu.VMEM_SHARED((_N, _F), jnp.float32),
        [pltpu.SemaphoreType.DMA] * 2,
        [pltpu.SemaphoreType.DMA] * 2,
        [pltpu.SemaphoreType.DMA] * 2,
    ],
)
def _sc_message_pass(h_hbm, src_hbm, dst_hbm, wf_hbm, zeros_hbm, out_hbm,
                     src_all, dst_v, rows_v, wf_v, acc,
                     semi, semg, sems):
    c = lax.axis_index("c")
    s = lax.axis_index("s")
    wid = c * _NUM_SUBCORES + s
    tile_base = wid * _EDGES_PER_TILE

    stripe = pl.ds(s * _ROWS_PER_TILE, _ROWS_PER_TILE)
    rem = pl.ds(_NUM_SUBCORES * _ROWS_PER_TILE, _ROWS_REMAINDER)
    pltpu.sync_copy(zeros_hbm.at[stripe], acc.at[stripe])

    @pl.when(s == 0)
    def _():
        pltpu.sync_copy(zeros_hbm.at[rem], acc.at[rem])

    pltpu.sync_copy(src_hbm.at[pl.ds(tile_base, _EDGES_PER_TILE)], src_all)
    plsc.subcore_barrier()

    def idx_copy(j, slot):
        return pltpu.make_async_copy(
            dst_hbm.at[pl.ds(tile_base + j * _CHUNK, _CHUNK)],
            dst_v[slot], semi[slot])

    def gather_copy(j, slot):
        return pltpu.make_async_copy(
            h_hbm.at[src_all.at[pl.ds(j * _CHUNK, _CHUNK)]],
            rows_v[slot], semg[slot])

    def wf_copy(j, slot):
        return pltpu.make_async_copy(
            wf_hbm.at[pl.ds(tile_base + j * _CHUNK, _CHUNK)],
            wf_v[slot], semg[slot])

    def scatter_copy(slot):
        return pltpu.make_async_copy(
            rows_v[slot], acc.at[dst_v[slot]], sems[slot])

    def issue(j, slot):
        idx_copy(j, slot).start()
        gather_copy(j, slot).start()
        wf_copy(j, slot).start()

    def multiply(slot):
        def mul_row(r, carry2):
            for cc in range(_F // 16):
                sl = pl.ds(cc * 16, 16)
                rows_v[slot][r, sl] = rows_v[slot][r, sl] * wf_v[slot][r, sl]
            return carry2

        lax.fori_loop(0, _CHUNK, mul_row, 0, unroll=False)

    def finish(j, slot):
        gather_copy(j, slot).wait()
        wf_copy(j, slot).wait()
        multiply(slot)
        idx_copy(j, slot).wait()
        pltpu.async_copy(rows_v[slot], acc.at[dst_v[slot]], sems[slot],
                         add=True)

    issue(0, 0)

    def pair_body(i, carry):
        for b in range(2):
            j = 2 * i + b
            @pl.when(j > 0)
            def _():
                scatter_copy(1 - b).wait()
            issue_j = j + 1
            idx_copy(issue_j, 1 - b).start()
            gather_copy(issue_j, 1 - b).start()
            wf_copy(issue_j, 1 - b).start()
            finish(j, b)
        return carry

    lax.fori_loop(0, _NCHUNKS // 2 - 1, pair_body, 0, unroll=False)
    scatter_copy(1).wait()
    idx_copy(_NCHUNKS - 1, 1).start()
    gather_copy(_NCHUNKS - 1, 1).start()
    wf_copy(_NCHUNKS - 1, 1).start()
    finish(_NCHUNKS - 2, 0)
    finish(_NCHUNKS - 1, 1)
    scatter_copy(0).wait()
    scatter_copy(1).wait()

    plsc.subcore_barrier()
    pltpu.sync_copy(acc.at[stripe], out_hbm.at[c, stripe])

    @pl.when(s == 0)
    def _():
        pltpu.sync_copy(acc.at[rem], out_hbm.at[c, rem])


def _out_body(p0_ref, p1_ref, w2_ref, b2_ref, w_ref, b_ref, o_ref):
    agg = p0_ref[...] + p1_ref[...]
    h2 = jnp.dot(agg, w2_ref[...], preferred_element_type=jnp.float32)
    h2 = h2 + b2_ref[...]
    o_ref[...] = jnp.dot(_ssp(h2), w_ref[...],
                         preferred_element_type=jnp.float32) + b_ref[...]


def _compute_out(p0, p1, lin2_w, lin2_b, lin_w, lin_b):
    nb = 10
    bn = _N // nb
    return pl.pallas_call(
        _out_body,
        grid=(nb,),
        in_specs=[
            pl.BlockSpec((bn, _F), lambda i: (i, 0)),
            pl.BlockSpec((bn, _F), lambda i: (i, 0)),
            pl.BlockSpec((_F, _F), lambda i: (0, 0)),
            pl.BlockSpec((1, _F), lambda i: (0, 0)),
            pl.BlockSpec((_F, _F), lambda i: (0, 0)),
            pl.BlockSpec((1, _F), lambda i: (0, 0)),
        ],
        out_specs=pl.BlockSpec((bn, _F), lambda i: (i, 0)),
        out_shape=jax.ShapeDtypeStruct((_N, _F), jnp.float32),
    )(p0, p1, lin2_w, lin2_b.reshape(1, _F), lin_w, lin_b.reshape(1, _F))


def kernel(x, edge_index, edge_weight, edge_attr, output,
           mlp_w1, mlp_b1, mlp_w2, mlp_b2,
           lin1_w, lin2_w, lin2_b, lin_w, lin_b):
    h = _compute_h(x, lin1_w)
    cutoff = _compute_cutoff(edge_weight)
    wf = _compute_wf(edge_attr.T, cutoff, mlp_w1, mlp_b1, mlp_w2, mlp_b2)
    src = edge_index[0]
    dst = edge_index[1]
    partials = _sc_message_pass(h, src, dst, wf, output)
    return _compute_out(partials[0], partials[1], lin2_w, lin2_b,
                        lin_w, lin_b)


# speedup vs baseline: 1.5150x; 1.3379x over previous
import functools

import jax
import jax.numpy as jnp
import numpy as np
from jax import lax
from jax.experimental import pallas as pl
from jax.experimental.pallas import tpu as pltpu
from jax.experimental.pallas import tpu_sc as plsc

_N = 10000
_E = 320000
_F = 128
_G = 50
_CUTOFF = 6.0

_NUM_CORES = 2
_NUM_SUBCORES = 16
_NUM_TILES = _NUM_CORES * _NUM_SUBCORES
_EDGES_PER_TILE = _E // _NUM_TILES
_CHUNK = 40
_NCHUNKS = _EDGES_PER_TILE // _CHUNK
_ROWS_PER_TILE = 624
_ROWS_REMAINDER = _N - _NUM_SUBCORES * _ROWS_PER_TILE


def _ssp(v):
    return jnp.maximum(v, 0.0) + jnp.log1p(jnp.exp(-jnp.abs(v))) - jnp.log(2.0)


def _h_body(x_ref, w_ref, o_ref):
    o_ref[...] = jnp.dot(x_ref[...], w_ref[...],
                         preferred_element_type=jnp.float32)


def _compute_h(x, lin1_w):
    nb = 10
    bn = _N // nb
    return pl.pallas_call(
        _h_body,
        grid=(nb,),
        in_specs=[
            pl.BlockSpec((bn, _F), lambda i: (i, 0)),
            pl.BlockSpec((_F, _F), lambda i: (0, 0)),
        ],
        out_specs=pl.BlockSpec((bn, _F), lambda i: (i, 0)),
        out_shape=jax.ShapeDtypeStruct((_N, _F), jnp.float32),
    )(x, lin1_w)


def _cutoff_body(ew_ref, o_ref):
    o_ref[...] = 0.5 * (jnp.cos(ew_ref[...] * (jnp.pi / _CUTOFF)) + 1.0)


def _compute_cutoff(edge_weight):
    ew2 = edge_weight.reshape(_E // _F, _F)
    return pl.pallas_call(
        _cutoff_body,
        out_shape=jax.ShapeDtypeStruct((_E // _F, _F), jnp.float32),
    )(ew2)


def _filter_body(ea_ref, c_ref, w1_ref, b1_ref, w2_ref, b2_ref, o_ref):
    a = jax.lax.dot_general(ea_ref[...], w1_ref[...],
                            (((0,), (0,)), ((), ())),
                            preferred_element_type=jnp.float32)
    a = a + b1_ref[...]
    f = jnp.dot(_ssp(a), w2_ref[...], preferred_element_type=jnp.float32)
    f = f + b2_ref[...]
    cmat = c_ref[0]
    ct = jnp.transpose(cmat)
    for s in range(cmat.shape[0]):
        rs = pl.ds(s * _F, _F)
        o_ref[rs, :] = f[s * _F:(s + 1) * _F, :] * ct[:, s:s + 1]


def _compute_wf(edge_attr_t, cutoff8, mlp_w1, mlp_b1, mlp_w2, mlp_b2):
    be = 2560
    nb = _E // be
    return pl.pallas_call(
        _filter_body,
        grid=(nb,),
        in_specs=[
            pl.BlockSpec((_G, be), lambda i: (0, i)),
            pl.BlockSpec((1, be // _F, _F), lambda i: (i, 0, 0)),
            pl.BlockSpec((_G, _F), lambda i: (0, 0)),
            pl.BlockSpec((1, _F), lambda i: (0, 0)),
            pl.BlockSpec((_F, _F), lambda i: (0, 0)),
            pl.BlockSpec((1, _F), lambda i: (0, 0)),
        ],
        out_specs=pl.BlockSpec((be, _F), lambda i: (i, 0)),
        out_shape=jax.ShapeDtypeStruct((_E, _F), jnp.float32),
    )(edge_attr_t, cutoff8.reshape(_E // be, be // _F, _F), mlp_w1,
      mlp_b1.reshape(1, _F), mlp_w2, mlp_b2.reshape(1, _F))


@functools.partial(
    pl.kernel,
    out_type=jax.ShapeDtypeStruct((_NUM_CORES, _N, _F), jnp.float32),
    mesh=plsc.VectorSubcoreMesh(core_axis_name="c", subcore_axis_name="s"),
    scratch_types=[
        pltpu.VMEM((_EDGES_PER_TILE,), jnp.int32),
        [pltpu.VMEM((_CHUNK,), jnp.int32)] * 2,
        [pltpu.VMEM((_CHUNK, _F), jnp.float32)] * 2,
        [pltpu.VMEM((_CHUNK, _F), jnp.float32)] * 2,
        pltpu.VMEM_SHARED((_N, _F), jnp.float32),
        [pltpu.SemaphoreType.DMA] * 2,
        [pltpu.SemaphoreType.DMA] * 2,
        [pltpu.SemaphoreType.DMA] * 2,
    ],
)
def _sc_message_pass(h_hbm, src_hbm, dst_hbm, wf_hbm, zeros_hbm, out_hbm,
                     src_all, dst_v, rows_v, wf_v, acc,
                     semi, semg, sems):
    c = lax.axis_index("c")
    s = lax.axis_index("s")
    wid = c * _NUM_SUBCORES + s
    tile_base = wid * _EDGES_PER_TILE

    stripe = pl.ds(s * _ROWS_PER_TILE, _ROWS_PER_TILE)
    rem = pl.ds(_NUM_SUBCORES * _ROWS_PER_TILE, _ROWS_REMAINDER)
    pltpu.sync_copy(zeros_hbm.at[stripe], acc.at[stripe])

    @pl.when(s == 0)
    def _():
        pltpu.sync_copy(zeros_hbm.at[rem], acc.at[rem])

    pltpu.sync_copy(src_hbm.at[pl.ds(tile_base, _EDGES_PER_TILE)], src_all)
    plsc.subcore_barrier()

    def idx_copy(j, slot):
        return pltpu.make_async_copy(
            dst_hbm.at[pl.ds(tile_base + j * _CHUNK, _CHUNK)],
            dst_v[slot], semi[slot])

    def gather_copy(j, slot):
        return pltpu.make_async_copy(
            h_hbm.at[src_all.at[pl.ds(j * _CHUNK, _CHUNK)]],
            rows_v[slot], semg[slot])

    def wf_copy(j, slot):
        return pltpu.make_async_copy(
            wf_hbm.at[pl.ds(tile_base + j * _CHUNK, _CHUNK)],
            wf_v[slot], semg[slot])

    def scatter_copy(slot):
        return pltpu.make_async_copy(
            rows_v[slot], acc.at[dst_v[slot]], sems[slot])

    def issue(j, slot):
        idx_copy(j, slot).start()
        gather_copy(j, slot).start()
        wf_copy(j, slot).start()

    def multiply(slot):
        def mul_row(r, carry2):
            for cc in range(_F // 16):
                sl = pl.ds(cc * 16, 16)
                rows_v[slot][r, sl] = rows_v[slot][r, sl] * wf_v[slot][r, sl]
            return carry2

        lax.fori_loop(0, _CHUNK, mul_row, 0, unroll=False)

    def finish(j, slot):
        gather_copy(j, slot).wait()
        wf_copy(j, slot).wait()
        multiply(slot)
        idx_copy(j, slot).wait()
        pltpu.async_copy(rows_v[slot], acc.at[dst_v[slot]], sems[slot],
                         add=True)

    issue(0, 0)

    def pair_body(i, carry):
        for b in range(2):
            j = 2 * i + b
            @pl.when(j > 0)
            def _():
                scatter_copy(1 - b).wait()
            issue_j = j + 1
            idx_copy(issue_j, 1 - b).start()
            gather_copy(issue_j, 1 - b).start()
            wf_copy(issue_j, 1 - b).start()
            finish(j, b)
        return carry

    lax.fori_loop(0, _NCHUNKS // 2 - 1, pair_body, 0, unroll=False)
    scatter_copy(1).wait()
    idx_copy(_NCHUNKS - 1, 1).start()
    gather_copy(_NCHUNKS - 1, 1).start()
    wf_copy(_NCHUNKS - 1, 1).start()
    finish(_NCHUNKS - 2, 0)
    finish(_NCHUNKS - 1, 1)
    scatter_copy(0).wait()
    scatter_copy(1).wait()

    plsc.subcore_barrier()
    pltpu.sync_copy(acc.at[stripe], out_hbm.at[c, stripe])

    @pl.when(s == 0)
    def _():
        pltpu.sync_copy(acc.at[rem], out_hbm.at[c, rem])


def _out_body(p0_ref, p1_ref, w2_ref, b2_ref, w_ref, b_ref, o_ref):
    agg = p0_ref[...] + p1_ref[...]
    h2 = jnp.dot(agg, w2_ref[...], preferred_element_type=jnp.float32)
    h2 = h2 + b2_ref[...]
    o_ref[...] = jnp.dot(_ssp(h2), w_ref[...],
                         preferred_element_type=jnp.float32) + b_ref[...]


def _compute_out(p0, p1, lin2_w, lin2_b, lin_w, lin_b):
    nb = 10
    bn = _N // nb
    return pl.pallas_call(
        _out_body,
        grid=(nb,),
        in_specs=[
            pl.BlockSpec((bn, _F), lambda i: (i, 0)),
            pl.BlockSpec((bn, _F), lambda i: (i, 0)),
            pl.BlockSpec((_F, _F), lambda i: (0, 0)),
            pl.BlockSpec((1, _F), lambda i: (0, 0)),
            pl.BlockSpec((_F, _F), lambda i: (0, 0)),
            pl.BlockSpec((1, _F), lambda i: (0, 0)),
        ],
        out_specs=pl.BlockSpec((bn, _F), lambda i: (i, 0)),
        out_shape=jax.ShapeDtypeStruct((_N, _F), jnp.float32),
    )(p0, p1, lin2_w, lin2_b.reshape(1, _F), lin_w, lin_b.reshape(1, _F))


def kernel(x, edge_index, edge_weight, edge_attr, output,
           mlp_w1, mlp_b1, mlp_w2, mlp_b2,
           lin1_w, lin2_w, lin2_b, lin_w, lin_b):
    h = _compute_h(x, lin1_w)
    cutoff = _compute_cutoff(edge_weight)
    wf = _compute_wf(edge_attr.T, cutoff, mlp_w1, mlp_b1, mlp_w2, mlp_b2)
    src = edge_index[0]
    dst = edge_index[1]
    partials = _sc_message_pass(h, src, dst, wf, output)
    return _compute_out(partials[0], partials[1], lin2_w, lin2_b,
                        lin_w, lin_b)


# speedup vs baseline: 1.6679x; 1.1009x over previous
import functools

import jax
import jax.numpy as jnp
import numpy as np
from jax import lax
from jax.experimental import pallas as pl
from jax.experimental.pallas import tpu as pltpu
from jax.experimental.pallas import tpu_sc as plsc

_N = 10000
_E = 320000
_F = 128
_G = 50
_CUTOFF = 6.0

_NUM_CORES = 2
_NUM_SUBCORES = 16
_NUM_TILES = _NUM_CORES * _NUM_SUBCORES
_EDGES_PER_TILE = _E // _NUM_TILES
_CHUNK = 40
_NCHUNKS = _EDGES_PER_TILE // _CHUNK
_ROWS_PER_TILE = 624
_ROWS_REMAINDER = _N - _NUM_SUBCORES * _ROWS_PER_TILE


def _ssp(v):
    return jnp.maximum(v, 0.0) + jnp.log1p(jnp.exp(-jnp.abs(v))) - jnp.log(2.0)


def _h_body(x_ref, w_ref, o_ref):
    o_ref[...] = jnp.dot(x_ref[...], w_ref[...],
                         preferred_element_type=jnp.float32)


def _compute_h(x, lin1_w):
    nb = 10
    bn = _N // nb
    return pl.pallas_call(
        _h_body,
        grid=(nb,),
        in_specs=[
            pl.BlockSpec((bn, _F), lambda i: (i, 0)),
            pl.BlockSpec((_F, _F), lambda i: (0, 0)),
        ],
        out_specs=pl.BlockSpec((bn, _F), lambda i: (i, 0)),
        out_shape=jax.ShapeDtypeStruct((_N, _F), jnp.float32),
    )(x, lin1_w)


def _cutoff_body(ew_ref, o_ref):
    o_ref[...] = 0.5 * (jnp.cos(ew_ref[...] * (jnp.pi / _CUTOFF)) + 1.0)


def _compute_cutoff(edge_weight):
    ew2 = edge_weight.reshape(_E // _F, _F)
    return pl.pallas_call(
        _cutoff_body,
        out_shape=jax.ShapeDtypeStruct((_E // _F, _F), jnp.float32),
    )(ew2)


def _filter_body(ea_ref, c_ref, w1_ref, b1_ref, w2_ref, b2_ref, o_ref):
    a = jax.lax.dot_general(ea_ref[...], w1_ref[...],
                            (((0,), (0,)), ((), ())),
                            preferred_element_type=jnp.float32)
    a = a + b1_ref[...]
    f = jnp.dot(_ssp(a), w2_ref[...], preferred_element_type=jnp.float32)
    f = f + b2_ref[...]
    cmat = c_ref[0]
    ct = jnp.transpose(cmat)
    for s in range(cmat.shape[0]):
        rs = pl.ds(s * _F, _F)
        o_ref[rs, :] = f[s * _F:(s + 1) * _F, :] * ct[:, s:s + 1]


_NSLICE = 5
_ES = _E // _NSLICE


def _compute_wf(edge_attr_t, cutoff3d, mlp_w1, mlp_b1, mlp_w2, mlp_b2, s):
    be = 2560
    nb = _ES // be
    return pl.pallas_call(
        _filter_body,
        grid=(nb,),
        in_specs=[
            pl.BlockSpec((_G, be), lambda i, s=s: (0, i + nb * s)),
            pl.BlockSpec((1, be // _F, _F), lambda i, s=s: (i + nb * s, 0, 0)),
            pl.BlockSpec((_G, _F), lambda i: (0, 0)),
            pl.BlockSpec((1, _F), lambda i: (0, 0)),
            pl.BlockSpec((_F, _F), lambda i: (0, 0)),
            pl.BlockSpec((1, _F), lambda i: (0, 0)),
        ],
        out_specs=pl.BlockSpec((be, _F), lambda i: (i, 0)),
        out_shape=jax.ShapeDtypeStruct((_ES, _F), jnp.float32),
    )(edge_attr_t, cutoff3d, mlp_w1,
      mlp_b1.reshape(1, _F), mlp_w2, mlp_b2.reshape(1, _F))


_EPT_S = _ES // _NUM_TILES
_NCHUNKS_S = _EPT_S // _CHUNK


@functools.partial(
    pl.kernel,
    out_type=jax.ShapeDtypeStruct((_NUM_CORES, _N, _F), jnp.float32),
    mesh=plsc.VectorSubcoreMesh(core_axis_name="c", subcore_axis_name="s"),
    scratch_types=[
        pltpu.VMEM((_EPT_S,), jnp.int32),
        [pltpu.VMEM((_CHUNK,), jnp.int32)] * 2,
        [pltpu.VMEM((_CHUNK, _F), jnp.float32)] * 2,
        [pltpu.VMEM((_CHUNK, _F), jnp.float32)] * 2,
        pltpu.VMEM_SHARED((_N, _F), jnp.float32),
        [pltpu.SemaphoreType.DMA] * 2,
        [pltpu.SemaphoreType.DMA] * 2,
        [pltpu.SemaphoreType.DMA] * 2,
    ],
)
def _sc_message_pass(h_hbm, src_hbm, dst_hbm, wf_hbm, prev_hbm, out_hbm,
                     src_all, dst_v, rows_v, wf_v, acc,
                     semi, semg, sems):
    c = lax.axis_index("c")
    s = lax.axis_index("s")
    wid = c * _NUM_SUBCORES + s
    tile_base = wid * _EPT_S

    stripe = pl.ds(s * _ROWS_PER_TILE, _ROWS_PER_TILE)
    rem = pl.ds(_NUM_SUBCORES * _ROWS_PER_TILE, _ROWS_REMAINDER)
    pltpu.sync_copy(prev_hbm.at[c, stripe], acc.at[stripe])

    @pl.when(s == 0)
    def _():
        pltpu.sync_copy(prev_hbm.at[c, rem], acc.at[rem])

    pltpu.sync_copy(src_hbm.at[pl.ds(tile_base, _EPT_S)], src_all)
    plsc.subcore_barrier()

    def idx_copy(j, slot):
        return pltpu.make_async_copy(
            dst_hbm.at[pl.ds(tile_base + j * _CHUNK, _CHUNK)],
            dst_v[slot], semi[slot])

    def gather_copy(j, slot):
        return pltpu.make_async_copy(
            h_hbm.at[src_all.at[pl.ds(j * _CHUNK, _CHUNK)]],
            rows_v[slot], semg[slot])

    def wf_copy(j, slot):
        return pltpu.make_async_copy(
            wf_hbm.at[pl.ds(tile_base + j * _CHUNK, _CHUNK)],
            wf_v[slot], semg[slot])

    def scatter_copy(slot):
        return pltpu.make_async_copy(
            rows_v[slot], acc.at[dst_v[slot]], sems[slot])

    def issue(j, slot):
        idx_copy(j, slot).start()
        gather_copy(j, slot).start()
        wf_copy(j, slot).start()

    def multiply(slot):
        def mul_row(r, carry2):
            for cc in range(_F // 16):
                sl = pl.ds(cc * 16, 16)
                rows_v[slot][r, sl] = rows_v[slot][r, sl] * wf_v[slot][r, sl]
            return carry2

        lax.fori_loop(0, _CHUNK, mul_row, 0, unroll=False)

    def finish(j, slot):
        gather_copy(j, slot).wait()
        wf_copy(j, slot).wait()
        multiply(slot)
        idx_copy(j, slot).wait()
        pltpu.async_copy(rows_v[slot], acc.at[dst_v[slot]], sems[slot],
                         add=True)

    issue(0, 0)

    def pair_body(i, carry):
        for b in range(2):
            j = 2 * i + b
            @pl.when(j > 0)
            def _():
                scatter_copy(1 - b).wait()
            issue_j = j + 1
            idx_copy(issue_j, 1 - b).start()
            gather_copy(issue_j, 1 - b).start()
            wf_copy(issue_j, 1 - b).start()
            finish(j, b)
        return carry

    lax.fori_loop(0, _NCHUNKS_S // 2 - 1, pair_body, 0, unroll=False)
    scatter_copy(1).wait()
    idx_copy(_NCHUNKS_S - 1, 1).start()
    gather_copy(_NCHUNKS_S - 1, 1).start()
    wf_copy(_NCHUNKS_S - 1, 1).start()
    finish(_NCHUNKS_S - 2, 0)
    finish(_NCHUNKS_S - 1, 1)
    scatter_copy(0).wait()
    scatter_copy(1).wait()

    plsc.subcore_barrier()
    pltpu.sync_copy(acc.at[stripe], out_hbm.at[c, stripe])

    @pl.when(s == 0)
    def _():
        pltpu.sync_copy(acc.at[rem], out_hbm.at[c, rem])


def _out_body(p0_ref, p1_ref, w2_ref, b2_ref, w_ref, b_ref, o_ref):
    agg = p0_ref[...] + p1_ref[...]
    h2 = jnp.dot(agg, w2_ref[...], preferred_element_type=jnp.float32)
    h2 = h2 + b2_ref[...]
    o_ref[...] = jnp.dot(_ssp(h2), w_ref[...],
                         preferred_element_type=jnp.float32) + b_ref[...]


def _compute_out(p0, p1, lin2_w, lin2_b, lin_w, lin_b):
    nb = 10
    bn = _N // nb
    return pl.pallas_call(
        _out_body,
        grid=(nb,),
        in_specs=[
            pl.BlockSpec((bn, _F), lambda i: (i, 0)),
            pl.BlockSpec((bn, _F), lambda i: (i, 0)),
            pl.BlockSpec((_F, _F), lambda i: (0, 0)),
            pl.BlockSpec((1, _F), lambda i: (0, 0)),
            pl.BlockSpec((_F, _F), lambda i: (0, 0)),
            pl.BlockSpec((1, _F), lambda i: (0, 0)),
        ],
        out_specs=pl.BlockSpec((bn, _F), lambda i: (i, 0)),
        out_shape=jax.ShapeDtypeStruct((_N, _F), jnp.float32),
    )(p0, p1, lin2_w, lin2_b.reshape(1, _F), lin_w, lin_b.reshape(1, _F))


def kernel(x, edge_index, edge_weight, edge_attr, output,
           mlp_w1, mlp_b1, mlp_w2, mlp_b2,
           lin1_w, lin2_w, lin2_b, lin_w, lin_b):
    h = _compute_h(x, lin1_w)
    cutoff = _compute_cutoff(edge_weight)
    cutoff3d = cutoff.reshape(_E // 2560, 2560 // _F, _F)
    ea_t = edge_attr.T
    src = edge_index[0]
    dst = edge_index[1]
    partials = jnp.zeros((_NUM_CORES, _N, _F), jnp.float32)
    for s in range(_NSLICE):
        wf_s = _compute_wf(ea_t, cutoff3d, mlp_w1, mlp_b1, mlp_w2,
                           mlp_b2, s)
        sl = slice(s * _ES, (s + 1) * _ES)
        partials = _sc_message_pass(h, src[sl], dst[sl], wf_s, partials)
    return _compute_out(partials[0], partials[1], lin2_w, lin2_b,
                        lin_w, lin_b)
